# baseline (speedup 1.0000x reference)
import math

import numpy as np
import jax
import jax.numpy as jnp
from jax.experimental import pallas as pl
from jax.experimental.pallas import tpu as pltpu

L = 2048
D_MODEL = 768
N_HEADS = 12
D_HEAD = 64
D_FF = 3072
N_LAYERS = 2
FACTOR = 5
U_TOP = min(int(FACTOR * math.ceil(math.log(L))), L)
BM = 512
_DN_T = (((1,), (1,)), ((), ()))


def _pos_embedding_np(seq_len, d_model):
    pe = np.zeros((seq_len, d_model), dtype=np.float32)
    position = np.arange(seq_len, dtype=np.float32)[:, None]
    div_term = np.exp(np.arange(0, d_model, 2, dtype=np.float32)
                      * -(math.log(10000.0) / d_model))
    pe[:, 0::2] = np.sin(position * div_term)
    pe[:, 1::2] = np.cos(position * div_term)
    return pe


def _sample_constants_np():
    out = []
    cpu = jax.devices("cpu")[0]
    with jax.default_device(cpu):
        for layer in range(N_LAYERS):
            skey = jax.random.key(1000 + layer)
            idx = np.asarray(jax.random.randint(skey, (L, U_TOP), 0, L))
            cnt = np.zeros((L, L), dtype=np.float32)
            np.add.at(cnt, (idx, np.arange(L)[:, None]), 1.0)
            neg = np.where(cnt > 0, 0.0, -1e30).astype(np.float32)
            out.append((cnt / L, neg))
    return out


_PE = _pos_embedding_np(L, D_MODEL)
try:
    _CONSTS = _sample_constants_np()
except Exception:
    _CONSTS = None


def _sample_constants(layer):
    if _CONSTS is not None:
        ps, neg = _CONSTS[layer]
        return jnp.asarray(ps), jnp.asarray(neg)
    skey = jax.random.key(1000 + layer)
    idx = jax.random.randint(skey, (L, U_TOP), 0, L)
    cnt = jnp.zeros((L, L), jnp.float32)
    cnt = cnt.at[idx, jnp.arange(L)[None, :].T].add(1.0)
    return cnt / L, jnp.where(cnt > 0, 0.0, -1e30).astype(jnp.float32)


def _mmT(x, w_ref, b_ref):
    return jax.lax.dot_general(x.astype(w_ref.dtype), w_ref[...], _DN_T,
                               preferred_element_type=jnp.float32) + b_ref[...]


def _ln(x, g_ref, b_ref):
    m = jnp.mean(x, axis=-1, keepdims=True)
    d = x - m
    v = jnp.mean(d * d, axis=-1, keepdims=True)
    return d / jnp.sqrt(v + 1e-5) * g_ref[...] + b_ref[...]


def _gelu(x):
    return 0.5 * x * (1.0 + jax.lax.erf(x * np.float32(1.0 / math.sqrt(2.0))))


def _enc_chain(ctx, xres, wo, bo, g1, b1, wc1, bc1, wc2, bc2, g2, b2):
    x1 = _ln(xres[...] + _mmT(ctx, wo, bo), g1, b1)
    y = _gelu(_mmT(x1, wc1, bc1))
    return _ln(x1 + _mmT(y, wc2, bc2), g2, b2)


def _d0_body(xcat_ref, pe_ref, wcat_ref, wq_ref, wk_ref, wv_ref,
             bq_ref, bk_ref, bv_ref, x_ref, q_ref, k_ref, v_ref):
    x = jax.lax.dot_general(xcat_ref[...], wcat_ref[...], _DN_T,
                            preferred_element_type=jnp.float32) + pe_ref[...]
    x_ref[...] = x
    q_ref[...] = _mmT(x, wq_ref, bq_ref)
    k_ref[...] = _mmT(x, wk_ref, bk_ref)
    v_ref[...] = _mmT(x, wv_ref, bv_ref)


def _d1_body(ctx_ref, xres_ref, wo_ref, bo_ref, g1_ref, b1_ref,
             wc1_ref, bc1_ref, wc2_ref, bc2_ref, g2_ref, b2_ref,
             wq_ref, wk_ref, wv_ref, bq_ref, bk_ref, bv_ref,
             x_ref, q_ref, k_ref, v_ref):
    x2 = _enc_chain(ctx_ref[...], xres_ref, wo_ref, bo_ref, g1_ref, b1_ref,
                    wc1_ref, bc1_ref, wc2_ref, bc2_ref, g2_ref, b2_ref)
    x_ref[...] = x2
    q_ref[...] = _mmT(x2, wq_ref, bq_ref)
    k_ref[...] = _mmT(x2, wk_ref, bk_ref)
    v_ref[...] = _mmT(x2, wv_ref, bv_ref)


def _d2_body(ctx_ref, xres_ref, wo_ref, bo_ref, g1_ref, b1_ref,
             wc1_ref, bc1_ref, wc2_ref, bc2_ref, g2_ref, b2_ref,
             ng_ref, nb_ref, wp_ref, bp_ref, out_ref):
    x2 = _enc_chain(ctx_ref[...], xres_ref, wo_ref, bo_ref, g1_ref, b1_ref,
                    wc1_ref, bc1_ref, wc2_ref, bc2_ref, g2_ref, b2_ref)
    out_ref[...] = _mmT(_ln(x2, ng_ref, nb_ref), wp_ref, bp_ref)


def _attm_body(q_ref, k_ref, ps_ref, neg_ref, m_ref, st_ref):
    for hh in range(2):
        cs = slice(D_HEAD * hh, D_HEAD * (hh + 1))
        st_ref[...] = jax.lax.dot_general(k_ref[:, cs].astype(jnp.bfloat16),
                                          q_ref[:, cs].astype(jnp.bfloat16),
                                          _DN_T,
                                          preferred_element_type=jnp.float32)
        st = st_ref[...]
        m_max = jnp.max(st + neg_ref[...].astype(jnp.float32),
                        axis=0, keepdims=True)
        m_sum = jnp.sum(st * ps_ref[...], axis=0, keepdims=True)
        m_ref[hh] = m_max - m_sum


def _attn_measure(q, k, ps, neg):
    neg = neg.astype(jnp.bfloat16)
    pair = pl.BlockSpec((L, 2 * D_HEAD), lambda j: (0, j))
    full = pl.BlockSpec((L, L), lambda j: (0, 0))
    return pl.pallas_call(
        _attm_body,
        grid=(N_HEADS // 2,),
        in_specs=[pair, pair, full, full],
        out_specs=pl.BlockSpec((2, 1, L), lambda j: (j, 0, 0)),
        out_shape=jax.ShapeDtypeStruct((N_HEADS, 1, L), jnp.float32),
        scratch_shapes=[pltpu.VMEM((L, L), jnp.float32)],
        compiler_params=pltpu.CompilerParams(
            dimension_semantics=("parallel",)),
    )(q, k, ps, neg)


HS = 6


def _atts_body(q_ref, k_ref, v_ref, m_ref, o_ref, *oh_refs):
    m0 = m_ref[...].reshape(HS, L)
    lane = jax.lax.broadcasted_iota(jnp.int32, (HS, L), 1)

    def pick(i, m_cur):
        top = jnp.max(m_cur, axis=1, keepdims=True)
        pos = jnp.min(jnp.where(m_cur == top, lane, L + 1), axis=1,
                      keepdims=True)
        sel = (lane == pos)
        self32 = sel.astype(jnp.float32)
        for hh in range(HS):
            oh_refs[hh][pl.ds(i, 1), :] = self32[hh:hh + 1, :]
        return jnp.where(sel, -1e30, m_cur)

    jax.lax.fori_loop(0, U_TOP, pick, m0)

    for hh in range(HS):
        cs = slice(D_HEAD * hh, D_HEAD * (hh + 1))
        q = q_ref[:, cs]
        k = k_ref[:, cs]
        v = v_ref[:, cs]
        o = oh_refs[hh][...]
        q_red = jax.lax.dot_general(o, q, (((1,), (0,)), ((), ())),
                                    preferred_element_type=jnp.float32)
        sc = jax.lax.dot_general(q_red, k, _DN_T,
                                 preferred_element_type=jnp.float32)
        sc = sc * np.float32(1.0 / math.sqrt(D_HEAD))
        sc = sc - jnp.max(sc, axis=1, keepdims=True)
        e = jnp.exp(sc)
        attn = e / jnp.sum(e, axis=1, keepdims=True)
        upd = jax.lax.dot_general(attn, v, (((1,), (0,)), ((), ())),
                                  preferred_element_type=jnp.float32)
        mean_v = jnp.mean(v, axis=0, keepdims=True)
        ctx = jax.lax.dot_general(o, upd, (((0,), (0,)), ((), ())),
                                  preferred_element_type=jnp.float32)
        rowsel = jax.lax.dot_general(o, jnp.ones((U_TOP, 1), jnp.float32),
                                     (((0,), (0,)), ((), ())),
                                     preferred_element_type=jnp.float32)
        o_ref[:, cs] = ctx + (1.0 - rowsel) * mean_v


def _attn_select(q, k, v, m):
    grp = pl.BlockSpec((L, HS * D_HEAD), lambda j: (0, j))
    return pl.pallas_call(
        _atts_body,
        grid=(N_HEADS // HS,),
        in_specs=[grp, grp, grp,
                  pl.BlockSpec((HS, 1, L), lambda j: (j, 0, 0))],
        out_specs=grp,
        out_shape=jax.ShapeDtypeStruct((L, D_MODEL), jnp.float32),
        scratch_shapes=[pltpu.VMEM((U_TOP, L), jnp.float32)
                        for _ in range(HS)],
        compiler_params=pltpu.CompilerParams(
            dimension_semantics=("parallel",)),
    )(q, k, v, m)


def _prob_attn(q, k, v, ps, neg):
    return _attn_select(q, k, v, _attn_measure(q, k, ps, neg))


def _row(n):
    return pl.BlockSpec((BM, n), lambda i: (i, 0))


def _const(n, k):
    return pl.BlockSpec((n, k), lambda i: (0, 0))


def _vec(n):
    return pl.BlockSpec((1, n), lambda i: (0, 0))


def _bf(w):
    return w.astype(jnp.bfloat16)


def _r1(b):
    return b.reshape(1, -1)


def kernel(batch_x, batch_x_time_stamp, batch_y, batch_y_time_stamp,
           batch_c, params):
    x_in = batch_x[0]
    mark = batch_x_time_stamp[0]
    enc_in = x_in.shape[1]
    mark_dim = mark.shape[1]

    kdim = 3 * enc_in + mark_dim
    kpad = 32
    x_cat = jnp.concatenate(
        [jnp.roll(x_in, 1, axis=0), x_in, jnp.roll(x_in, -1, axis=0), mark],
        axis=-1)
    x_cat = jnp.pad(x_cat, ((0, 0), (0, kpad - kdim)))
    wv = params['Wv_emb']
    w_cat = jnp.concatenate([wv[:, :, 0], wv[:, :, 1], wv[:, :, 2],
                             params['Wt_emb']], axis=1)
    w_cat = jnp.pad(w_cat, ((0, 0), (0, kpad - kdim)))
    pe = jnp.asarray(_PE)

    p0, p1 = params['layers'][0], params['layers'][1]
    d = D_MODEL
    xs = jax.ShapeDtypeStruct((L, d), jnp.float32)
    wspec = _const(d, d)
    c1spec, c2spec = _const(D_FF, d), _const(d, D_FF)

    x, q, k, v = pl.pallas_call(
        _d0_body,
        grid=(L // BM,),
        in_specs=[_row(kpad), _row(d), _const(d, kpad),
                  wspec, wspec, wspec, _vec(d), _vec(d), _vec(d)],
        out_specs=(_row(d), _row(d), _row(d), _row(d)),
        out_shape=(xs, xs, xs, xs),
        compiler_params=pltpu.CompilerParams(
            dimension_semantics=("parallel",)),
    )(x_cat, pe, w_cat, _bf(p0['Wq']), _bf(p0['Wk']), _bf(p0['Wv']),
      _r1(p0['bq']), _r1(p0['bk']), _r1(p0['bv']))

    ps0, neg0 = _sample_constants(0)
    ctx = _prob_attn(q, k, v, ps0, neg0)

    x, q, k, v = pl.pallas_call(
        _d1_body,
        grid=(L // BM,),
        in_specs=[_row(d), _row(d), wspec, _vec(d), _vec(d), _vec(d),
                  c1spec, _vec(D_FF), c2spec, _vec(d), _vec(d), _vec(d),
                  wspec, wspec, wspec, _vec(d), _vec(d), _vec(d)],
        out_specs=(_row(d), _row(d), _row(d), _row(d)),
        out_shape=(xs, xs, xs, xs),
        compiler_params=pltpu.CompilerParams(
            dimension_semantics=("parallel",)),
    )(ctx, x, _bf(p0['Wo']), _r1(p0['bo']), _r1(p0['g1']), _r1(p0['b1']),
      _bf(p0['Wc1']), _r1(p0['bc1']), _bf(p0['Wc2']), _r1(p0['bc2']),
      _r1(p0['g2']), _r1(p0['b2']),
      _bf(p1['Wq']), _bf(p1['Wk']), _bf(p1['Wv']),
      _r1(p1['bq']), _r1(p1['bk']), _r1(p1['bv']))

    ps1, neg1 = _sample_constants(1)
    ctx = _prob_attn(q, k, v, ps1, neg1)

    c_out = params['Wproj'].shape[0]
    npad = 128
    wp = jnp.pad(params['Wproj'], ((0, npad - c_out), (0, 0)))
    bp = jnp.pad(params['bproj'], (0, npad - c_out))
    out = pl.pallas_call(
        _d2_body,
        grid=(L // BM,),
        in_specs=[_row(d), _row(d), wspec, _vec(d), _vec(d), _vec(d),
                  c1spec, _vec(D_FF), c2spec, _vec(d), _vec(d), _vec(d),
                  _vec(d), _vec(d), _const(npad, d), _vec(npad)],
        out_specs=_row(npad),
        out_shape=jax.ShapeDtypeStruct((L, npad), jnp.float32),
        compiler_params=pltpu.CompilerParams(
            dimension_semantics=("parallel",)),
    )(ctx, x, _bf(p1['Wo']), _r1(p1['bo']), _r1(p1['g1']), _r1(p1['b1']),
      _bf(p1['Wc1']), _r1(p1['bc1']), _bf(p1['Wc2']), _r1(p1['bc2']),
      _r1(p1['g2']), _r1(p1['b2']),
      _r1(params['norm_g']), _r1(params['norm_b']), wp, _r1(bp))
    return out[:, :c_out][None]


# speedup vs baseline: 1.0563x; 1.0563x over previous
import math

import numpy as np
import jax
import jax.numpy as jnp
from jax.experimental import pallas as pl
from jax.experimental.pallas import tpu as pltpu

L = 2048
D_MODEL = 768
N_HEADS = 12
D_HEAD = 64
D_FF = 3072
N_LAYERS = 2
FACTOR = 5
U_TOP = min(int(FACTOR * math.ceil(math.log(L))), L)
BM = 512
_DN_T = (((1,), (1,)), ((), ()))


def _pos_embedding_np(seq_len, d_model):
    pe = np.zeros((seq_len, d_model), dtype=np.float32)
    position = np.arange(seq_len, dtype=np.float32)[:, None]
    div_term = np.exp(np.arange(0, d_model, 2, dtype=np.float32)
                      * -(math.log(10000.0) / d_model))
    pe[:, 0::2] = np.sin(position * div_term)
    pe[:, 1::2] = np.cos(position * div_term)
    return pe


def _sample_constants_np():
    out = []
    cpu = jax.devices("cpu")[0]
    with jax.default_device(cpu):
        for layer in range(N_LAYERS):
            skey = jax.random.key(1000 + layer)
            idx = np.asarray(jax.random.randint(skey, (L, U_TOP), 0, L))
            cnt = np.zeros((L, L), dtype=np.float32)
            np.add.at(cnt, (idx, np.arange(L)[:, None]), 1.0)
            neg = np.where(cnt > 0, 0.0, -1e30).astype(np.float32)
            out.append((cnt / L, neg))
    return out


_PE = _pos_embedding_np(L, D_MODEL)
try:
    _CONSTS = _sample_constants_np()
except Exception:
    _CONSTS = None


def _sample_constants(layer):
    if _CONSTS is not None:
        ps, neg = _CONSTS[layer]
        return jnp.asarray(ps), jnp.asarray(neg)
    skey = jax.random.key(1000 + layer)
    idx = jax.random.randint(skey, (L, U_TOP), 0, L)
    cnt = jnp.zeros((L, L), jnp.float32)
    cnt = cnt.at[idx, jnp.arange(L)[None, :].T].add(1.0)
    return cnt / L, jnp.where(cnt > 0, 0.0, -1e30).astype(jnp.float32)


def _mmT(x, w_ref, b_ref):
    return jax.lax.dot_general(x.astype(jnp.bfloat16),
                               w_ref[...].astype(jnp.bfloat16), _DN_T,
                               preferred_element_type=jnp.float32) + b_ref[...]


def _ln(x, g_ref, b_ref):
    m = jnp.mean(x, axis=-1, keepdims=True)
    d = x - m
    v = jnp.mean(d * d, axis=-1, keepdims=True)
    return d / jnp.sqrt(v + 1e-5) * g_ref[...] + b_ref[...]


def _gelu(x):
    return 0.5 * x * (1.0 + jax.lax.erf(x * np.float32(1.0 / math.sqrt(2.0))))


def _enc_chain(ctx, xres, wo, bo, g1, b1, wc1, bc1, wc2, bc2, g2, b2):
    x1 = _ln(xres[...] + _mmT(ctx, wo, bo), g1, b1)
    y = _gelu(_mmT(x1, wc1, bc1))
    return _ln(x1 + _mmT(y, wc2, bc2), g2, b2)


def _d0_body(xcat_ref, pe_ref, wcat_ref, wq_ref, wk_ref, wv_ref,
             bq_ref, bk_ref, bv_ref, x_ref, q_ref, k_ref, v_ref):
    x = jax.lax.dot_general(xcat_ref[...], wcat_ref[...], _DN_T,
                            preferred_element_type=jnp.float32) + pe_ref[...]
    x_ref[...] = x
    q_ref[...] = _mmT(x, wq_ref, bq_ref)
    k_ref[...] = _mmT(x, wk_ref, bk_ref)
    v_ref[...] = _mmT(x, wv_ref, bv_ref)


def _d1_body(ctx_ref, xres_ref, wo_ref, bo_ref, g1_ref, b1_ref,
             wc1_ref, bc1_ref, wc2_ref, bc2_ref, g2_ref, b2_ref,
             wq_ref, wk_ref, wv_ref, bq_ref, bk_ref, bv_ref,
             x_ref, q_ref, k_ref, v_ref):
    x2 = _enc_chain(ctx_ref[...], xres_ref, wo_ref, bo_ref, g1_ref, b1_ref,
                    wc1_ref, bc1_ref, wc2_ref, bc2_ref, g2_ref, b2_ref)
    x_ref[...] = x2
    q_ref[...] = _mmT(x2, wq_ref, bq_ref)
    k_ref[...] = _mmT(x2, wk_ref, bk_ref)
    v_ref[...] = _mmT(x2, wv_ref, bv_ref)


def _d2_body(ctx_ref, xres_ref, wo_ref, bo_ref, g1_ref, b1_ref,
             wc1_ref, bc1_ref, wc2_ref, bc2_ref, g2_ref, b2_ref,
             ng_ref, nb_ref, wp_ref, bp_ref, out_ref):
    x2 = _enc_chain(ctx_ref[...], xres_ref, wo_ref, bo_ref, g1_ref, b1_ref,
                    wc1_ref, bc1_ref, wc2_ref, bc2_ref, g2_ref, b2_ref)
    xf = _ln(x2, ng_ref, nb_ref)
    out_ref[...] = jax.lax.dot_general(
        xf, wp_ref[...], _DN_T,
        preferred_element_type=jnp.float32) + bp_ref[...]


def _attm_body(q_ref, k_ref, ps_ref, neg_ref, m_ref, st_ref):
    for hh in range(2):
        cs = slice(D_HEAD * hh, D_HEAD * (hh + 1))
        st_ref[...] = jax.lax.dot_general(k_ref[:, cs], q_ref[:, cs], _DN_T,
                                          preferred_element_type=jnp.float32)
        st = st_ref[...]
        m_max = jnp.max(st + neg_ref[...].astype(jnp.float32),
                        axis=0, keepdims=True)
        m_sum = jnp.sum(st * ps_ref[...], axis=0, keepdims=True)
        m_ref[hh] = m_max - m_sum


def _attn_measure(q, k, ps, neg):
    neg = neg.astype(jnp.bfloat16)
    pair = pl.BlockSpec((L, 2 * D_HEAD), lambda j: (0, j))
    full = pl.BlockSpec((L, L), lambda j: (0, 0))
    return pl.pallas_call(
        _attm_body,
        grid=(N_HEADS // 2,),
        in_specs=[pair, pair, full, full],
        out_specs=pl.BlockSpec((2, 1, L), lambda j: (j, 0, 0)),
        out_shape=jax.ShapeDtypeStruct((N_HEADS, 1, L), jnp.float32),
        scratch_shapes=[pltpu.VMEM((L, L), jnp.float32)],
        compiler_params=pltpu.CompilerParams(
            dimension_semantics=("parallel",)),
    )(q, k, ps, neg)


HS = 6


def _atts_body(q_ref, k_ref, v_ref, m_ref, o_ref, *oh_refs):
    m0 = m_ref[...].reshape(HS, L)
    lane = jax.lax.broadcasted_iota(jnp.int32, (HS, L), 1)

    def pick(i, m_cur):
        top = jnp.max(m_cur, axis=1, keepdims=True)
        pos = jnp.min(jnp.where(m_cur == top, lane, L + 1), axis=1,
                      keepdims=True)
        sel = (lane == pos)
        self32 = sel.astype(jnp.float32)
        for hh in range(HS):
            oh_refs[hh][pl.ds(i, 1), :] = self32[hh:hh + 1, :]
        return jnp.where(sel, -1e30, m_cur)

    jax.lax.fori_loop(0, U_TOP, pick, m0)

    for hh in range(HS):
        cs = slice(D_HEAD * hh, D_HEAD * (hh + 1))
        q = q_ref[:, cs]
        k = k_ref[:, cs]
        v = v_ref[:, cs]
        o = oh_refs[hh][...]
        q_red = jax.lax.dot_general(o, q, (((1,), (0,)), ((), ())),
                                    preferred_element_type=jnp.float32)
        sc = jax.lax.dot_general(q_red, k, _DN_T,
                                 preferred_element_type=jnp.float32)
        sc = sc * np.float32(1.0 / math.sqrt(D_HEAD))
        sc = sc - jnp.max(sc, axis=1, keepdims=True)
        e = jnp.exp(sc)
        attn = e / jnp.sum(e, axis=1, keepdims=True)
        upd = jax.lax.dot_general(attn, v, (((1,), (0,)), ((), ())),
                                  preferred_element_type=jnp.float32)
        mean_v = jnp.mean(v, axis=0, keepdims=True)
        ctx = jax.lax.dot_general(o, upd, (((0,), (0,)), ((), ())),
                                  preferred_element_type=jnp.float32)
        rowsel = jax.lax.dot_general(o, jnp.ones((U_TOP, 1), jnp.float32),
                                     (((0,), (0,)), ((), ())),
                                     preferred_element_type=jnp.float32)
        o_ref[:, cs] = ctx + (1.0 - rowsel) * mean_v


def _attn_select(q, k, v, m):
    grp = pl.BlockSpec((L, HS * D_HEAD), lambda j: (0, j))
    return pl.pallas_call(
        _atts_body,
        grid=(N_HEADS // HS,),
        in_specs=[grp, grp, grp,
                  pl.BlockSpec((HS, 1, L), lambda j: (j, 0, 0))],
        out_specs=grp,
        out_shape=jax.ShapeDtypeStruct((L, D_MODEL), jnp.float32),
        scratch_shapes=[pltpu.VMEM((U_TOP, L), jnp.float32)
                        for _ in range(HS)],
        compiler_params=pltpu.CompilerParams(
            dimension_semantics=("parallel",)),
    )(q, k, v, m)


def _prob_attn(q, k, v, ps, neg):
    return _attn_select(q, k, v, _attn_measure(q, k, ps, neg))


def _row(n):
    return pl.BlockSpec((BM, n), lambda i: (i, 0))


def _const(n, k):
    return pl.BlockSpec((n, k), lambda i: (0, 0))


def _vec(n):
    return pl.BlockSpec((1, n), lambda i: (0, 0))


def _bf(w):
    return w


def _r1(b):
    return b.reshape(1, -1)


def kernel(batch_x, batch_x_time_stamp, batch_y, batch_y_time_stamp,
           batch_c, params):
    x_in = batch_x[0]
    mark = batch_x_time_stamp[0]
    enc_in = x_in.shape[1]
    mark_dim = mark.shape[1]

    kdim = 3 * enc_in + mark_dim
    kpad = 32
    x_cat = jnp.concatenate(
        [jnp.roll(x_in, 1, axis=0), x_in, jnp.roll(x_in, -1, axis=0), mark],
        axis=-1)
    x_cat = jnp.pad(x_cat, ((0, 0), (0, kpad - kdim)))
    wv = params['Wv_emb']
    w_cat = jnp.concatenate([wv[:, :, 0], wv[:, :, 1], wv[:, :, 2],
                             params['Wt_emb']], axis=1)
    w_cat = jnp.pad(w_cat, ((0, 0), (0, kpad - kdim)))
    pe = jnp.asarray(_PE)

    p0, p1 = params['layers'][0], params['layers'][1]
    d = D_MODEL
    xs = jax.ShapeDtypeStruct((L, d), jnp.float32)
    wspec = _const(d, d)
    c1spec, c2spec = _const(D_FF, d), _const(d, D_FF)

    x, q, k, v = pl.pallas_call(
        _d0_body,
        grid=(L // BM,),
        in_specs=[_row(kpad), _row(d), _const(d, kpad),
                  wspec, wspec, wspec, _vec(d), _vec(d), _vec(d)],
        out_specs=(_row(d), _row(d), _row(d), _row(d)),
        out_shape=(xs, xs, xs, xs),
        compiler_params=pltpu.CompilerParams(
            dimension_semantics=("parallel",)),
    )(x_cat, pe, w_cat, _bf(p0['Wq']), _bf(p0['Wk']), _bf(p0['Wv']),
      _r1(p0['bq']), _r1(p0['bk']), _r1(p0['bv']))

    ps0, neg0 = _sample_constants(0)
    ctx = _prob_attn(q, k, v, ps0, neg0)

    x, q, k, v = pl.pallas_call(
        _d1_body,
        grid=(L // BM,),
        in_specs=[_row(d), _row(d), wspec, _vec(d), _vec(d), _vec(d),
                  c1spec, _vec(D_FF), c2spec, _vec(d), _vec(d), _vec(d),
                  wspec, wspec, wspec, _vec(d), _vec(d), _vec(d)],
        out_specs=(_row(d), _row(d), _row(d), _row(d)),
        out_shape=(xs, xs, xs, xs),
        compiler_params=pltpu.CompilerParams(
            dimension_semantics=("parallel",)),
    )(ctx, x, _bf(p0['Wo']), _r1(p0['bo']), _r1(p0['g1']), _r1(p0['b1']),
      _bf(p0['Wc1']), _r1(p0['bc1']), _bf(p0['Wc2']), _r1(p0['bc2']),
      _r1(p0['g2']), _r1(p0['b2']),
      _bf(p1['Wq']), _bf(p1['Wk']), _bf(p1['Wv']),
      _r1(p1['bq']), _r1(p1['bk']), _r1(p1['bv']))

    ps1, neg1 = _sample_constants(1)
    ctx = _prob_attn(q, k, v, ps1, neg1)

    c_out = params['Wproj'].shape[0]
    npad = 128
    wp = jnp.pad(params['Wproj'], ((0, npad - c_out), (0, 0)))
    bp = jnp.pad(params['bproj'], (0, npad - c_out))
    out = pl.pallas_call(
        _d2_body,
        grid=(L // BM,),
        in_specs=[_row(d), _row(d), wspec, _vec(d), _vec(d), _vec(d),
                  c1spec, _vec(D_FF), c2spec, _vec(d), _vec(d), _vec(d),
                  _vec(d), _vec(d), _const(npad, d), _vec(npad)],
        out_specs=_row(npad),
        out_shape=jax.ShapeDtypeStruct((L, npad), jnp.float32),
        compiler_params=pltpu.CompilerParams(
            dimension_semantics=("parallel",)),
    )(ctx, x, _bf(p1['Wo']), _r1(p1['bo']), _r1(p1['g1']), _r1(p1['b1']),
      _bf(p1['Wc1']), _r1(p1['bc1']), _bf(p1['Wc2']), _r1(p1['bc2']),
      _r1(p1['g2']), _r1(p1['b2']),
      _r1(params['norm_g']), _r1(params['norm_b']), wp, _r1(bp))
    return out[:, :c_out][None]


# speedup vs baseline: 1.0630x; 1.0063x over previous
import math

import numpy as np
import jax
import jax.numpy as jnp
from jax.experimental import pallas as pl
from jax.experimental.pallas import tpu as pltpu

L = 2048
D_MODEL = 768
N_HEADS = 12
D_HEAD = 64
D_FF = 3072
N_LAYERS = 2
FACTOR = 5
U_TOP = min(int(FACTOR * math.ceil(math.log(L))), L)
BM = 512
_DN_T = (((1,), (1,)), ((), ()))


def _pos_embedding_np(seq_len, d_model):
    pe = np.zeros((seq_len, d_model), dtype=np.float32)
    position = np.arange(seq_len, dtype=np.float32)[:, None]
    div_term = np.exp(np.arange(0, d_model, 2, dtype=np.float32)
                      * -(math.log(10000.0) / d_model))
    pe[:, 0::2] = np.sin(position * div_term)
    pe[:, 1::2] = np.cos(position * div_term)
    return pe


def _sample_constants_np():
    out = []
    cpu = jax.devices("cpu")[0]
    with jax.default_device(cpu):
        for layer in range(N_LAYERS):
            skey = jax.random.key(1000 + layer)
            idx = np.asarray(jax.random.randint(skey, (L, U_TOP), 0, L))
            cnt = np.zeros((L, L), dtype=np.float32)
            np.add.at(cnt, (idx, np.arange(L)[:, None]), 1.0)
            neg = np.where(cnt > 0, 0.0, -1e30).astype(np.float32)
            out.append((cnt / L, neg))
    return out


_PE = _pos_embedding_np(L, D_MODEL)
try:
    _CONSTS = _sample_constants_np()
except Exception:
    _CONSTS = None


def _sample_constants(layer):
    if _CONSTS is not None:
        ps, neg = _CONSTS[layer]
        return jnp.asarray(ps), jnp.asarray(neg)
    skey = jax.random.key(1000 + layer)
    idx = jax.random.randint(skey, (L, U_TOP), 0, L)
    cnt = jnp.zeros((L, L), jnp.float32)
    cnt = cnt.at[idx, jnp.arange(L)[None, :].T].add(1.0)
    return cnt / L, jnp.where(cnt > 0, 0.0, -1e30).astype(jnp.float32)


def _mmT(x, w_ref, b_ref):
    return jax.lax.dot_general(x.astype(jnp.bfloat16),
                               w_ref[...].astype(jnp.bfloat16), _DN_T,
                               preferred_element_type=jnp.float32) + b_ref[...]


def _ln(x, g_ref, b_ref):
    m = jnp.mean(x, axis=-1, keepdims=True)
    d = x - m
    v = jnp.mean(d * d, axis=-1, keepdims=True)
    return d / jnp.sqrt(v + 1e-5) * g_ref[...] + b_ref[...]


def _gelu(x):
    return 0.5 * x * (1.0 + jax.lax.erf(x * np.float32(1.0 / math.sqrt(2.0))))


def _enc_chain(ctx, xres, wo, bo, g1, b1, wc1, bc1, wc2, bc2, g2, b2):
    x1 = _ln(xres[...] + _mmT(ctx, wo, bo), g1, b1)
    y = _gelu(_mmT(x1, wc1, bc1))
    return _ln(x1 + _mmT(y, wc2, bc2), g2, b2)


def _d0_body(xcat_ref, pe_ref, wcat_ref, wq_ref, wk_ref, wv_ref,
             bq_ref, bk_ref, bv_ref, x_ref, q_ref, k_ref, v_ref):
    x = jax.lax.dot_general(xcat_ref[...], wcat_ref[...], _DN_T,
                            preferred_element_type=jnp.float32) + pe_ref[...]
    x_ref[...] = x
    q_ref[...] = _mmT(x, wq_ref, bq_ref)
    k_ref[...] = _mmT(x, wk_ref, bk_ref)
    v_ref[...] = _mmT(x, wv_ref, bv_ref)


def _d1_body(ctx_ref, xres_ref, wo_ref, bo_ref, g1_ref, b1_ref,
             wc1_ref, bc1_ref, wc2_ref, bc2_ref, g2_ref, b2_ref,
             wq_ref, wk_ref, wv_ref, bq_ref, bk_ref, bv_ref,
             x_ref, q_ref, k_ref, v_ref):
    x2 = _enc_chain(ctx_ref[...], xres_ref, wo_ref, bo_ref, g1_ref, b1_ref,
                    wc1_ref, bc1_ref, wc2_ref, bc2_ref, g2_ref, b2_ref)
    x_ref[...] = x2
    q_ref[...] = _mmT(x2, wq_ref, bq_ref)
    k_ref[...] = _mmT(x2, wk_ref, bk_ref)
    v_ref[...] = _mmT(x2, wv_ref, bv_ref)


def _d2_body(ctx_ref, xres_ref, wo_ref, bo_ref, g1_ref, b1_ref,
             wc1_ref, bc1_ref, wc2_ref, bc2_ref, g2_ref, b2_ref,
             ng_ref, nb_ref, wp_ref, bp_ref, out_ref):
    x2 = _enc_chain(ctx_ref[...], xres_ref, wo_ref, bo_ref, g1_ref, b1_ref,
                    wc1_ref, bc1_ref, wc2_ref, bc2_ref, g2_ref, b2_ref)
    xf = _ln(x2, ng_ref, nb_ref)
    out_ref[...] = jax.lax.dot_general(
        xf, wp_ref[...], _DN_T,
        preferred_element_type=jnp.float32) + bp_ref[...]


def _attm_body(q_ref, k_ref, ps_ref, neg_ref, m_ref, st_ref):
    for hh in range(2):
        cs = slice(D_HEAD * hh, D_HEAD * (hh + 1))
        st_ref[...] = jax.lax.dot_general(k_ref[:, cs], q_ref[:, cs], _DN_T,
                                          preferred_element_type=jnp.float32)
        st = st_ref[...]
        m_max = jnp.max(st + neg_ref[...].astype(jnp.float32),
                        axis=0, keepdims=True)
        m_sum = jnp.sum(st * ps_ref[...], axis=0, keepdims=True)
        m_ref[hh] = m_max - m_sum


def _attn_measure(q, k, ps, neg):
    neg = neg.astype(jnp.bfloat16)
    pair = pl.BlockSpec((L, 2 * D_HEAD), lambda j: (0, j))
    full = pl.BlockSpec((L, L), lambda j: (0, 0))
    return pl.pallas_call(
        _attm_body,
        grid=(N_HEADS // 2,),
        in_specs=[pair, pair, full, full],
        out_specs=pl.BlockSpec((2, 1, L), lambda j: (j, 0, 0)),
        out_shape=jax.ShapeDtypeStruct((N_HEADS, 1, L), jnp.float32),
        scratch_shapes=[pltpu.VMEM((L, L), jnp.float32)],
        compiler_params=pltpu.CompilerParams(
            dimension_semantics=("parallel",)),
    )(q, k, ps, neg)


HS = 6


def _atts_body(q_ref, k_ref, v_ref, m_ref, o_ref, *oh_refs):
    m0 = m_ref[...].reshape(HS, L)
    lane = jax.lax.broadcasted_iota(jnp.int32, (HS, L), 1)

    def pick(i, m_cur):
        top = jnp.max(m_cur, axis=1, keepdims=True)
        pos = jnp.min(jnp.where(m_cur == top, lane, L + 1), axis=1,
                      keepdims=True)
        sel = (lane == pos)
        self32 = sel.astype(jnp.float32)
        for hh in range(HS):
            oh_refs[hh][pl.ds(i, 1), :] = self32[hh:hh + 1, :]
        return jnp.where(sel, -1e30, m_cur)

    jax.lax.fori_loop(0, U_TOP, pick, m0)

    for hh in range(HS):
        cs = slice(D_HEAD * hh, D_HEAD * (hh + 1))
        q = q_ref[:, cs]
        k = k_ref[:, cs]
        v = v_ref[:, cs]
        o = oh_refs[hh][...]
        q_red = jax.lax.dot_general(o, q, (((1,), (0,)), ((), ())),
                                    preferred_element_type=jnp.float32)
        sc = jax.lax.dot_general(q_red, k, _DN_T,
                                 preferred_element_type=jnp.float32)
        sc = sc * np.float32(1.0 / math.sqrt(D_HEAD))
        sc = sc - jnp.max(sc, axis=1, keepdims=True)
        e = jnp.exp(sc)
        attn = e / jnp.sum(e, axis=1, keepdims=True)
        upd = jax.lax.dot_general(attn, v, (((1,), (0,)), ((), ())),
                                  preferred_element_type=jnp.float32)
        mean_v = jnp.mean(v, axis=0, keepdims=True)
        ctx = jax.lax.dot_general(o, upd, (((0,), (0,)), ((), ())),
                                  preferred_element_type=jnp.float32)
        rowsel = jax.lax.dot_general(o, jnp.ones((U_TOP, 1), jnp.float32),
                                     (((0,), (0,)), ((), ())),
                                     preferred_element_type=jnp.float32)
        o_ref[:, cs] = ctx + (1.0 - rowsel) * mean_v


def _attn_select(q, k, v, m):
    grp = pl.BlockSpec((L, HS * D_HEAD), lambda j: (0, j))
    return pl.pallas_call(
        _atts_body,
        grid=(N_HEADS // HS,),
        in_specs=[grp, grp, grp,
                  pl.BlockSpec((HS, 1, L), lambda j: (j, 0, 0))],
        out_specs=grp,
        out_shape=jax.ShapeDtypeStruct((L, D_MODEL), jnp.float32),
        scratch_shapes=[pltpu.VMEM((U_TOP, L), jnp.float32)
                        for _ in range(HS)],
        compiler_params=pltpu.CompilerParams(
            dimension_semantics=("parallel",)),
    )(q, k, v, m)


def _prob_attn(q, k, v, ps, neg):
    return _attn_select(q, k, v, _attn_measure(q, k, ps, neg))


def _row(n):
    return pl.BlockSpec((BM, n), lambda i: (i, 0))


def _const(n, k):
    return pl.BlockSpec((n, k), lambda i: (0, 0))


def _vec(n):
    return pl.BlockSpec((1, n), lambda i: (0, 0))


def _bf(w):
    return w


def _r1(b):
    return b.reshape(1, -1)


def kernel(batch_x, batch_x_time_stamp, batch_y, batch_y_time_stamp,
           batch_c, params):
    x_in = batch_x[0]
    mark = batch_x_time_stamp[0]
    enc_in = x_in.shape[1]
    mark_dim = mark.shape[1]

    kdim = 3 * enc_in + mark_dim
    x_cat = jnp.concatenate(
        [jnp.roll(x_in, 1, axis=0), x_in, jnp.roll(x_in, -1, axis=0), mark],
        axis=-1)
    wv = params['Wv_emb']
    w_cat = jnp.concatenate([wv[:, :, 0], wv[:, :, 1], wv[:, :, 2],
                             params['Wt_emb']], axis=1)
    pe = jnp.asarray(_PE)

    p0, p1 = params['layers'][0], params['layers'][1]
    d = D_MODEL
    xs = jax.ShapeDtypeStruct((L, d), jnp.float32)
    wspec = _const(d, d)
    c1spec, c2spec = _const(D_FF, d), _const(d, D_FF)

    x, q, k, v = pl.pallas_call(
        _d0_body,
        grid=(L // BM,),
        in_specs=[_row(kdim), _row(d), _const(d, kdim),
                  wspec, wspec, wspec, _vec(d), _vec(d), _vec(d)],
        out_specs=(_row(d), _row(d), _row(d), _row(d)),
        out_shape=(xs, xs, xs, xs),
        compiler_params=pltpu.CompilerParams(
            dimension_semantics=("parallel",)),
    )(x_cat, pe, w_cat, _bf(p0['Wq']), _bf(p0['Wk']), _bf(p0['Wv']),
      _r1(p0['bq']), _r1(p0['bk']), _r1(p0['bv']))

    ps0, neg0 = _sample_constants(0)
    ctx = _prob_attn(q, k, v, ps0, neg0)

    x, q, k, v = pl.pallas_call(
        _d1_body,
        grid=(L // BM,),
        in_specs=[_row(d), _row(d), wspec, _vec(d), _vec(d), _vec(d),
                  c1spec, _vec(D_FF), c2spec, _vec(d), _vec(d), _vec(d),
                  wspec, wspec, wspec, _vec(d), _vec(d), _vec(d)],
        out_specs=(_row(d), _row(d), _row(d), _row(d)),
        out_shape=(xs, xs, xs, xs),
        compiler_params=pltpu.CompilerParams(
            dimension_semantics=("parallel",)),
    )(ctx, x, _bf(p0['Wo']), _r1(p0['bo']), _r1(p0['g1']), _r1(p0['b1']),
      _bf(p0['Wc1']), _r1(p0['bc1']), _bf(p0['Wc2']), _r1(p0['bc2']),
      _r1(p0['g2']), _r1(p0['b2']),
      _bf(p1['Wq']), _bf(p1['Wk']), _bf(p1['Wv']),
      _r1(p1['bq']), _r1(p1['bk']), _r1(p1['bv']))

    ps1, neg1 = _sample_constants(1)
    ctx = _prob_attn(q, k, v, ps1, neg1)

    c_out = params['Wproj'].shape[0]
    out = pl.pallas_call(
        _d2_body,
        grid=(L // BM,),
        in_specs=[_row(d), _row(d), wspec, _vec(d), _vec(d), _vec(d),
                  c1spec, _vec(D_FF), c2spec, _vec(d), _vec(d), _vec(d),
                  _vec(d), _vec(d), _const(c_out, d), _vec(c_out)],
        out_specs=_row(c_out),
        out_shape=jax.ShapeDtypeStruct((L, c_out), jnp.float32),
        compiler_params=pltpu.CompilerParams(
            dimension_semantics=("parallel",)),
    )(ctx, x, _bf(p1['Wo']), _r1(p1['bo']), _r1(p1['g1']), _r1(p1['b1']),
      _bf(p1['Wc1']), _r1(p1['bc1']), _bf(p1['Wc2']), _r1(p1['bc2']),
      _r1(p1['g2']), _r1(p1['b2']),
      _r1(params['norm_g']), _r1(params['norm_b']), params['Wproj'],
      _r1(params['bproj']))
    return out[None]
